# baseline (speedup 1.0000x reference)
import functools

import jax
import jax.numpy as jnp
from jax import lax
from jax.experimental import pallas as pl
from jax.experimental.pallas import tpu as pltpu
from jax.experimental.pallas import tpu_sc as plsc

LOSS_COEF = 0.01
BM = 128
NW = 32


def _router_body(x_ref, wg_ref, gates1_ref, gates2_ref, pos1_ref, pos2_ref,
                 be_ref, loss_ref):
    x = x_ref[...]
    wg = wg_ref[...]
    n, e_dim = x.shape[0], wg.shape[1]
    logits = jnp.dot(x, wg, preferred_element_type=jnp.float32)
    eidx = jax.lax.broadcasted_iota(jnp.int32, (n, e_dim), 1)
    m1 = jnp.max(logits, axis=1, keepdims=True)
    a1 = jnp.min(jnp.where(logits == m1, eidx, e_dim), axis=1, keepdims=True)
    rest = jnp.where(eidx == a1, -jnp.inf, logits)
    m2 = jnp.max(rest, axis=1, keepdims=True)
    a2 = jnp.min(jnp.where(rest == m2, eidx, e_dim), axis=1, keepdims=True)
    z = jnp.exp(m2 - m1)
    denom = 1.0 + z
    g1 = 1.0 / denom
    g2 = z / denom
    oh1 = (eidx == a1).astype(jnp.float32)
    oh2 = (eidx == a2).astype(jnp.float32)
    gates = g1 * oh1 + g2 * oh2
    importance = jnp.sum(gates, axis=0)
    load = jnp.sum((gates > 0.0).astype(jnp.float32), axis=0)

    def cv2(v):
        m = jnp.mean(v)
        var = jnp.sum((v - m) ** 2) / (e_dim - 1)
        return var / (m * m + 1e-10)

    loss_ref[0, 0] = (cv2(importance) + cv2(load)) * LOSS_COEF

    tri = (jax.lax.broadcasted_iota(jnp.int32, (n, n), 0)
           > jax.lax.broadcasted_iota(jnp.int32, (n, n), 1)).astype(jnp.float32)
    rank1 = jnp.dot(tri, oh1, preferred_element_type=jnp.float32)
    cnt1 = jnp.sum(oh1, axis=0, keepdims=True)
    rank2 = jnp.dot(tri, oh2, preferred_element_type=jnp.float32) + cnt1
    cnt = cnt1 + jnp.sum(oh2, axis=0, keepdims=True)
    bmf = jnp.float32(BM)
    pcnt = jnp.floor((cnt + (bmf - 1.0)) / bmf) * bmf
    tri_e = (jax.lax.broadcasted_iota(jnp.int32, (e_dim, e_dim), 0)
             < jax.lax.broadcasted_iota(jnp.int32, (e_dim, e_dim), 1)
             ).astype(jnp.float32)
    pstart = jnp.dot(pcnt, tri_e, preferred_element_type=jnp.float32)
    pos1 = jnp.sum(jnp.where(eidx == a1, rank1 + pstart, 0.0), axis=1,
                   keepdims=True)
    pos2 = jnp.sum(jnp.where(eidx == a2, rank2 + pstart, 0.0), axis=1,
                   keepdims=True)
    pos1_ref[...] = pos1.astype(jnp.int32)
    pos2_ref[...] = pos2.astype(jnp.int32)
    gates1_ref[...] = jnp.broadcast_to(g1, (n, 128))
    gates2_ref[...] = jnp.broadcast_to(g2, (n, 128))
    nb = be_ref.shape[0]
    blk = jax.lax.broadcasted_iota(jnp.int32, (nb, e_dim), 0).astype(jnp.float32)
    psb = jnp.broadcast_to(pstart / bmf, (nb, e_dim))
    be = jnp.sum((psb <= blk).astype(jnp.int32), axis=1, keepdims=True) - 1
    be_ref[...] = be


def _make_dispatch(n_tok, d, s_slots):
    mesh = plsc.VectorSubcoreMesh(core_axis_name="c", subcore_axis_name="s")
    tchunk = (2 * n_tok) // NW
    half = tchunk // 2

    @functools.partial(
        pl.kernel, mesh=mesh,
        out_type=[
            jax.ShapeDtypeStruct((s_slots, d), jnp.float32),
            jax.ShapeDtypeStruct((s_slots, 128), jnp.float32),
        ],
        scratch_types=[
            pltpu.VMEM((2, half), jnp.int32),
            pltpu.VMEM((2, half, 128), jnp.float32),
            pltpu.VMEM((half, d), jnp.float32),
            pltpu.SemaphoreType.DMA,
            pltpu.SemaphoreType.DMA,
        ],
    )
    def dispatch(x_hbm, pos_hbm, g_hbm, xg_hbm, gs_hbm, pos_v, g_v, xbuf,
                 sem1, sem2):
        c = lax.axis_index("c")
        s_ = lax.axis_index("s")
        w = s_ * 2 + c
        t0 = (w % 16) * tchunk
        pltpu.sync_copy(pos_hbm.at[w], pos_v)
        pltpu.sync_copy(g_hbm.at[w], g_v)
        for j in range(2):
            pltpu.sync_copy(x_hbm.at[pl.ds(t0 + j * half, half)], xbuf)
            cp1 = pltpu.async_copy(xbuf, xg_hbm.at[pos_v.at[j]], sem1)
            cp2 = pltpu.async_copy(g_v.at[j], gs_hbm.at[pos_v.at[j]], sem2)
            cp1.wait()
            cp2.wait()

    return dispatch


def _gmm_body(be_ref, xg_ref, w_ref, b_ref, gs_ref, out_ref):
    i = pl.program_id(0)
    be = be_ref[i]
    acc = jnp.dot(xg_ref[...].astype(jnp.bfloat16),
                  w_ref[0].astype(jnp.bfloat16),
                  preferred_element_type=jnp.float32)
    ridx = jax.lax.broadcasted_iota(jnp.int32, b_ref.shape, 0)
    brow = jnp.sum(jnp.where(ridx == be, b_ref[...], 0.0), axis=0,
                   keepdims=True)
    out_ref[...] = (gs_ref[:, 0:1] * (acc + brow)).astype(jnp.bfloat16)


def _make_regroup(n_pairs, h2):
    mesh = plsc.VectorSubcoreMesh(core_axis_name="c", subcore_axis_name="s")
    per_w = n_pairs // NW
    sub = 32
    nsub = per_w // sub

    @functools.partial(
        pl.kernel, mesh=mesh,
        out_type=jax.ShapeDtypeStruct((n_pairs, h2), jnp.int32),
        scratch_types=[
            pltpu.VMEM((nsub, sub), jnp.int32),
            pltpu.VMEM((sub, h2), jnp.int32),
            pltpu.SemaphoreType.DMA,
        ],
    )
    def regroup(op_hbm, pos_hbm, cmb_hbm, pos_v, buf, sem):
        c = lax.axis_index("c")
        s_ = lax.axis_index("s")
        w = s_ * 2 + c
        p0 = w * per_w
        pltpu.sync_copy(pos_hbm.at[w], pos_v)
        for j in range(nsub):
            pltpu.async_copy(op_hbm.at[pos_v.at[j]], buf, sem).wait()
            pltpu.sync_copy(buf, cmb_hbm.at[pl.ds(p0 + j * sub, sub)])

    return regroup


def _add_body(a_ref, b_ref, out_ref):
    out_ref[...] = (a_ref[0].astype(jnp.float32)
                    + b_ref[0].astype(jnp.float32))


def kernel(x, w_gate, W, b):
    orig_shape = x.shape[:-1]
    d = x.shape[-1]
    xf = x.reshape(-1, d)
    n = xf.shape[0]
    e_dim, _, h = W.shape
    p_pairs = 2 * n
    s_slots = p_pairs + e_dim * BM
    nb = s_slots // BM

    g1, g2, pos1, pos2, be, loss = pl.pallas_call(
        _router_body,
        out_shape=[
            jax.ShapeDtypeStruct((n, 128), jnp.float32),
            jax.ShapeDtypeStruct((n, 128), jnp.float32),
            jax.ShapeDtypeStruct((n, 1), jnp.int32),
            jax.ShapeDtypeStruct((n, 1), jnp.int32),
            jax.ShapeDtypeStruct((128, 1), jnp.int32),
            jax.ShapeDtypeStruct((1, 1), jnp.float32),
        ],
        out_specs=[
            pl.BlockSpec(memory_space=pltpu.VMEM),
            pl.BlockSpec(memory_space=pltpu.VMEM),
            pl.BlockSpec(memory_space=pltpu.VMEM),
            pl.BlockSpec(memory_space=pltpu.VMEM),
            pl.BlockSpec(memory_space=pltpu.VMEM),
            pl.BlockSpec(memory_space=pltpu.SMEM),
        ],
    )(xf, w_gate)

    half = (p_pairs // NW) // 2
    pos_all = jnp.concatenate([pos1, pos2], axis=0)
    pos_w = pos_all.reshape(NW, 2, half)
    g_w = jnp.concatenate([g1, g2], axis=0).reshape(NW, 2, half, 128)
    pos_c = pos_all.reshape(NW, (p_pairs // NW) // 32, 32)

    xg, gs = _make_dispatch(n, d, s_slots)(xf, pos_w, g_w)

    grid_spec = pltpu.PrefetchScalarGridSpec(
        num_scalar_prefetch=1,
        grid=(nb,),
        in_specs=[
            pl.BlockSpec((BM, d), lambda i, be_r: (i, 0)),
            pl.BlockSpec((1, d, h), lambda i, be_r: (be_r[i], 0, 0)),
            pl.BlockSpec((e_dim, h), lambda i, be_r: (0, 0)),
            pl.BlockSpec((BM, 128), lambda i, be_r: (i, 0)),
        ],
        out_specs=pl.BlockSpec((BM, h), lambda i, be_r: (i, 0)),
    )
    out_pairs = pl.pallas_call(
        _gmm_body,
        grid_spec=grid_spec,
        out_shape=jax.ShapeDtypeStruct((s_slots, h), jnp.bfloat16),
        compiler_params=pltpu.CompilerParams(
            dimension_semantics=("arbitrary",),
        ),
    )(be.reshape(-1)[:nb], xg, W, b, gs)

    op_i32 = jax.lax.bitcast_convert_type(
        out_pairs.reshape(s_slots, h // 2, 2), jnp.int32)
    cmb_i32 = _make_regroup(p_pairs, h // 2)(op_i32, pos_c)
    cmb3 = jax.lax.bitcast_convert_type(
        cmb_i32, jnp.bfloat16).reshape(2, n, h)

    tn = 256
    y = pl.pallas_call(
        _add_body,
        grid=(n // tn,),
        in_specs=[
            pl.BlockSpec((1, tn, h), lambda t: (0, t, 0)),
            pl.BlockSpec((1, tn, h), lambda t: (1, t, 0)),
        ],
        out_specs=pl.BlockSpec((tn, h), lambda t: (t, 0)),
        out_shape=jax.ShapeDtypeStruct((n, h), jnp.float32),
    )(cmb3, cmb3)

    return y.reshape(orig_shape + (h,)), loss[0, 0]
